# Initial kernel scaffold; baseline (speedup 1.0000x reference)
import jax
import jax.numpy as jnp
from jax.experimental import pallas as pl


def kernel(x, batch, c1_W1, c1_b1, c1_g, c1_be, c1_W2, c1_b2, c2_W1, c2_b1, c2_g, c2_be, c2_W2, c2_b2, c3_W1, c3_b1, c3_g, c3_be, c3_W2, c3_b2, m_W1, m_b1, m_W2, m_b2, m_W3, m_b3, m_W4, m_b4):
    raise NotImplementedError("write your pallas kernel here")


# speedup vs baseline: 3.4256x; 3.4256x over previous
import functools

import jax
import jax.numpy as jnp
from jax import lax
from jax.experimental import pallas as pl
from jax.experimental.pallas import tpu as pltpu
from jax.experimental.pallas import tpu_sc as plsc

N = 10000
K = 16
H = 64
RB = 128
NP = 10112
DP = 128
NB = 200
NG = N // NB
E = N * K
CH = 128
EPAD = 163840
BIGF = 3.0e38
BIGI = 1 << 30


def _dist_body(hp_ref, ht_ref, wa_ref, wb_ref, vec_ref,
               idx_ref, u_ref, c_ref):
    hb = hp_ref[...]
    ht = ht_ref[...]
    d2 = -2.0 * jnp.dot(hb, ht, preferred_element_type=jnp.float32)
    sq_row = jnp.sum(hb * hb, axis=1, keepdims=True)
    sq_col = jnp.sum(ht * ht, axis=0, keepdims=True)
    d2 = d2 + sq_row + sq_col
    col = lax.broadcasted_iota(jnp.int32, (RB, NP), 1)
    d2 = jnp.where(col < N, d2, BIGF)
    picks = []
    for _ in range(K):
        vmin = jnp.min(d2, axis=1, keepdims=True)
        ismin = d2 <= vmin
        idxk = jnp.min(jnp.where(ismin, col, BIGI), axis=1, keepdims=True)
        picks.append(idxk)
        d2 = jnp.where(col == idxk, BIGF, d2)
    idx_ref[...] = jnp.concatenate(picks, axis=1)
    u = jnp.dot(hb, wb_ref[...], preferred_element_type=jnp.float32)
    u_ref[...] = u
    c_ref[...] = (jnp.dot(hb, wa_ref[...], preferred_element_type=jnp.float32)
                  + vec_ref[0:1, :] - u[:, 0:H])


def _stats_body(uj_ref, c_ref, stats_ref):
    first = (pl.program_id(0) == 0) & (pl.program_id(1) == 0)

    @pl.when(first)
    def _():
        stats_ref[...] = jnp.zeros((8, H), jnp.float32)

    m = uj_ref[:, 0:H] + c_ref[...]
    s1 = jnp.sum(m, axis=0, keepdims=True)
    s2 = jnp.sum(m * m, axis=0, keepdims=True)
    stats_ref[0:1, :] += s1
    stats_ref[1:2, :] += s2


def _out_body(uj_ref, c_ref, stats_ref, vec_ref, w2_ref, out_ref):
    k = pl.program_id(1)
    m = uj_ref[:, 0:H] + c_ref[...]
    mu = stats_ref[0:1, :] / jnp.float32(E)
    var = stats_ref[1:2, :] / jnp.float32(E) - mu * mu
    g = vec_ref[0:1, :]
    be = vec_ref[1:2, :]
    b2 = vec_ref[2:3, :]
    mnorm = g * (m - mu) / jnp.sqrt(var + 1e-5) + be
    y = jnp.maximum(mnorm, 0.0)
    y = jnp.dot(y, w2_ref[...], preferred_element_type=jnp.float32) + b2

    @pl.when(k == 0)
    def _():
        out_ref[...] = y

    @pl.when(k > 0)
    def _():
        out_ref[...] = jnp.maximum(out_ref[...], y)


def _mlp_body(x1_ref, x2_ref, x3_ref,
              w1_ref, b1_ref, w2_ref, b2_ref,
              w3_ref, b3_ref, w4_ref, b4_ref, out_ref):
    h = jnp.concatenate([x1_ref[...], x2_ref[...], x3_ref[...]], axis=1)
    h = jnp.maximum(jnp.dot(h, w1_ref[...], preferred_element_type=jnp.float32)
                    + b1_ref[0:1, :], 0.0)
    h = jnp.maximum(jnp.dot(h, w2_ref[...], preferred_element_type=jnp.float32)
                    + b2_ref[0:1, :], 0.0)
    h = jnp.maximum(jnp.dot(h, w3_ref[...], preferred_element_type=jnp.float32)
                    + b3_ref[0:1, :], 0.0)
    z = jnp.dot(h, w4_ref[...], preferred_element_type=jnp.float32) + b4_ref[0:1, :]
    zmax = jnp.max(z, axis=1, keepdims=True)
    ez = jnp.exp(z - zmax)
    lse = jnp.log(jnp.sum(ez, axis=1, keepdims=True)) + zmax
    out_ref[...] = z - lse


def _dist_call(hp, ht, wa, wb, vec):
    grid = NP // RB
    return pl.pallas_call(
        _dist_body,
        grid=(grid,),
        in_specs=[
            pl.BlockSpec((RB, DP), lambda i: (i, 0)),
            pl.BlockSpec((DP, NP), lambda i: (0, 0)),
            pl.BlockSpec((DP, H), lambda i: (0, 0)),
            pl.BlockSpec((DP, DP), lambda i: (0, 0)),
            pl.BlockSpec((8, H), lambda i: (0, 0)),
        ],
        out_specs=[
            pl.BlockSpec((RB, K), lambda i: (i, 0)),
            pl.BlockSpec((RB, DP), lambda i: (i, 0)),
            pl.BlockSpec((RB, H), lambda i: (i, 0)),
        ],
        out_shape=[
            jax.ShapeDtypeStruct((NP, K), jnp.int32),
            jax.ShapeDtypeStruct((NP, DP), jnp.float32),
            jax.ShapeDtypeStruct((NP, H), jnp.float32),
        ],
    )(hp, ht, wa, wb, vec)


def _stats_call(uj, c):
    return pl.pallas_call(
        _stats_body,
        grid=(K, NG),
        in_specs=[
            pl.BlockSpec((NB, DP), lambda k, n: (k * NG + n, 0)),
            pl.BlockSpec((NB, H), lambda k, n: (n, 0)),
        ],
        out_specs=pl.BlockSpec((8, H), lambda k, n: (0, 0)),
        out_shape=jax.ShapeDtypeStruct((8, H), jnp.float32),
    )(uj, c)


def _out_call(uj, c, stats, vec, w2):
    return pl.pallas_call(
        _out_body,
        grid=(NG, K),
        in_specs=[
            pl.BlockSpec((NB, DP), lambda n, k: (k * NG + n, 0)),
            pl.BlockSpec((NB, H), lambda n, k: (n, 0)),
            pl.BlockSpec((8, H), lambda n, k: (0, 0)),
            pl.BlockSpec((8, H), lambda n, k: (0, 0)),
            pl.BlockSpec((H, H), lambda n, k: (0, 0)),
        ],
        out_specs=pl.BlockSpec((NB, H), lambda n, k: (n, 0)),
        out_shape=jax.ShapeDtypeStruct((N, H), jnp.float32),
    )(uj, c, stats, vec, w2)


def _mlp_call(x1, x2, x3, w1, b1, w2, b2, w3, b3, w4, b4):
    mb = 400
    grid = N // mb
    row = lambda i: (i, 0)
    cst = lambda i: (0, 0)
    return pl.pallas_call(
        _mlp_body,
        grid=(grid,),
        in_specs=[
            pl.BlockSpec((mb, H), row),
            pl.BlockSpec((mb, H), row),
            pl.BlockSpec((mb, H), row),
            pl.BlockSpec((192, 128), cst),
            pl.BlockSpec((8, 128), cst),
            pl.BlockSpec((128, 64), cst),
            pl.BlockSpec((8, 64), cst),
            pl.BlockSpec((64, 32), cst),
            pl.BlockSpec((8, 32), cst),
            pl.BlockSpec((32, 2), cst),
            pl.BlockSpec((8, 2), cst),
        ],
        out_specs=pl.BlockSpec((mb, 2), row),
        out_shape=jax.ShapeDtypeStruct((N, 2), jnp.float32),
    )(x1, x2, x3, w1, b1, w2, b2, w3, b3, w4, b4)


def _make_sc_gather():
    info = plsc.get_sparse_core_info()
    nc = info.num_cores
    nw = nc * info.num_subcores
    chunks_per_w = EPAD // (nw * CH)
    mesh = plsc.VectorSubcoreMesh(core_axis_name="c", subcore_axis_name="s")

    @functools.partial(
        pl.kernel, mesh=mesh,
        out_type=jax.ShapeDtypeStruct((EPAD, DP), jnp.float32),
        scratch_types=[
            pltpu.VMEM((CH,), jnp.int32),
            pltpu.VMEM((CH, DP), jnp.float32),
            pltpu.SemaphoreType.DMA,
        ],
    )
    def gather(u_hbm, idx_hbm, out_hbm, idx_v, rows_v, sem):
        wid = lax.axis_index("s") * nc + lax.axis_index("c")

        def step(i, carry):
            off = (wid * chunks_per_w + i) * CH
            pltpu.sync_copy(idx_hbm.at[pl.ds(off, CH)], idx_v)
            pltpu.async_copy(u_hbm.at[idx_v], rows_v, sem).wait()
            pltpu.sync_copy(rows_v, out_hbm.at[pl.ds(off, CH)])
            return carry

        lax.fori_loop(0, chunks_per_w, step, 0)

    return gather


_sc_gather_cache = []


def _sc_gather(u, idx):
    if not _sc_gather_cache:
        _sc_gather_cache.append(_make_sc_gather())
    return _sc_gather_cache[0](u, idx)


def _vec8(*rows, width=H):
    v = jnp.zeros((8, width), jnp.float32)
    for r, x in enumerate(rows):
        v = v.at[r].set(x)
    return v


def _edge_layer(h, w1, b1, g, be, w2, b2):
    d = h.shape[1]
    wa = jnp.pad(w1[:d], ((0, DP - d), (0, 0)))
    wb = jnp.pad(w1[d:], ((0, DP - d), (0, DP - H)))
    hp = jnp.pad(h, ((0, NP - N), (0, DP - d)))
    ht = hp.T
    idxp, u_p, c_p = _dist_call(hp, ht, wa, wb, _vec8(b1))
    idx_km = idxp[:N].T.reshape(-1)
    idx_pad = jnp.pad(idx_km, (0, EPAD - E))
    uj = _sc_gather(u_p[:N], idx_pad)
    stats = _stats_call(uj, c_p)
    return _out_call(uj, c_p, stats, _vec8(g, be, b2), w2)


def kernel(x, batch, c1_W1, c1_b1, c1_g, c1_be, c1_W2, c1_b2,
           c2_W1, c2_b1, c2_g, c2_be, c2_W2, c2_b2,
           c3_W1, c3_b1, c3_g, c3_be, c3_W2, c3_b2,
           m_W1, m_b1, m_W2, m_b2, m_W3, m_b3, m_W4, m_b4):
    x1 = _edge_layer(x, c1_W1, c1_b1, c1_g, c1_be, c1_W2, c1_b2)
    x2 = _edge_layer(x1, c2_W1, c2_b1, c2_g, c2_be, c2_W2, c2_b2)
    x3 = _edge_layer(x2, c3_W1, c3_b1, c3_g, c3_be, c3_W2, c3_b2)
    return _mlp_call(x1, x2, x3,
                     m_W1, _vec8(m_b1, width=128),
                     m_W2, _vec8(m_b2, width=64),
                     m_W3, _vec8(m_b3, width=32),
                     m_W4, _vec8(m_b4, width=2))


# speedup vs baseline: 4.5796x; 1.3369x over previous
import functools

import jax
import jax.numpy as jnp
from jax import lax
from jax.experimental import pallas as pl
from jax.experimental.pallas import tpu as pltpu
from jax.experimental.pallas import tpu_sc as plsc

N = 10000
K = 16
H = 64
RB = 128
NP = 10112
DP = 128
NB = 2000
NG = N // NB
E = N * K
CH = 128
EPAD = 163840
BIGF = 3.0e38
BIGI = 1 << 30


def _dist_body(hp_ref, ht_ref, wa_ref, wb_ref, vec_ref,
               idx_ref, u_ref, c_ref):
    hb = hp_ref[...]
    ht = ht_ref[...]
    d2 = -2.0 * jnp.dot(hb, ht, preferred_element_type=jnp.float32)
    sq_row = jnp.sum(hb * hb, axis=1, keepdims=True)
    sq_col = jnp.sum(ht * ht, axis=0, keepdims=True)
    d2 = d2 + sq_row + sq_col
    col = lax.broadcasted_iota(jnp.int32, (RB, NP), 1)
    d2 = jnp.where(col < N, d2, BIGF)
    picks = []
    for _ in range(K):
        vmin = jnp.min(d2, axis=1, keepdims=True)
        ismin = d2 <= vmin
        idxk = jnp.min(jnp.where(ismin, col, BIGI), axis=1, keepdims=True)
        picks.append(idxk)
        d2 = jnp.where(col == idxk, BIGF, d2)
    idx_ref[...] = jnp.concatenate(picks, axis=1)
    u = jnp.dot(hb, wb_ref[...], preferred_element_type=jnp.float32)
    u_ref[...] = u
    c_ref[...] = (jnp.dot(hb, wa_ref[...], preferred_element_type=jnp.float32)
                  + vec_ref[0:1, :] - u[:, 0:H])


def _stats_body(uj_ref, c_ref, stats_ref):
    first = (pl.program_id(0) == 0) & (pl.program_id(1) == 0)

    @pl.when(first)
    def _():
        stats_ref[...] = jnp.zeros((8, H), jnp.float32)

    m = uj_ref[:, 0:H] + c_ref[...]
    s1 = jnp.sum(m, axis=0, keepdims=True)
    s2 = jnp.sum(m * m, axis=0, keepdims=True)
    stats_ref[0:1, :] += s1
    stats_ref[1:2, :] += s2


def _out_body(uj_ref, c_ref, stats_ref, vec_ref, w2_ref, out_ref):
    k = pl.program_id(1)
    m = uj_ref[:, 0:H] + c_ref[...]
    mu = stats_ref[0:1, :] / jnp.float32(E)
    var = stats_ref[1:2, :] / jnp.float32(E) - mu * mu
    g = vec_ref[0:1, :]
    be = vec_ref[1:2, :]
    b2 = vec_ref[2:3, :]
    mnorm = g * (m - mu) / jnp.sqrt(var + 1e-5) + be
    y = jnp.maximum(mnorm, 0.0)
    y = jnp.dot(y, w2_ref[...], preferred_element_type=jnp.float32) + b2

    @pl.when(k == 0)
    def _():
        out_ref[...] = y

    @pl.when(k > 0)
    def _():
        out_ref[...] = jnp.maximum(out_ref[...], y)


def _mlp_body(x1_ref, x2_ref, x3_ref,
              w1_ref, b1_ref, w2_ref, b2_ref,
              w3_ref, b3_ref, w4_ref, b4_ref, out_ref):
    h = jnp.concatenate([x1_ref[...], x2_ref[...], x3_ref[...]], axis=1)
    h = jnp.maximum(jnp.dot(h, w1_ref[...], preferred_element_type=jnp.float32)
                    + b1_ref[0:1, :], 0.0)
    h = jnp.maximum(jnp.dot(h, w2_ref[...], preferred_element_type=jnp.float32)
                    + b2_ref[0:1, :], 0.0)
    h = jnp.maximum(jnp.dot(h, w3_ref[...], preferred_element_type=jnp.float32)
                    + b3_ref[0:1, :], 0.0)
    z = jnp.dot(h, w4_ref[...], preferred_element_type=jnp.float32) + b4_ref[0:1, :]
    zmax = jnp.max(z, axis=1, keepdims=True)
    ez = jnp.exp(z - zmax)
    lse = jnp.log(jnp.sum(ez, axis=1, keepdims=True)) + zmax
    out_ref[...] = z - lse


def _dist_call(hp, ht, wa, wb, vec):
    grid = NP // RB
    return pl.pallas_call(
        _dist_body,
        grid=(grid,),
        in_specs=[
            pl.BlockSpec((RB, DP), lambda i: (i, 0)),
            pl.BlockSpec((DP, NP), lambda i: (0, 0)),
            pl.BlockSpec((DP, H), lambda i: (0, 0)),
            pl.BlockSpec((DP, DP), lambda i: (0, 0)),
            pl.BlockSpec((8, H), lambda i: (0, 0)),
        ],
        out_specs=[
            pl.BlockSpec((RB, K), lambda i: (i, 0)),
            pl.BlockSpec((RB, DP), lambda i: (i, 0)),
            pl.BlockSpec((RB, H), lambda i: (i, 0)),
        ],
        out_shape=[
            jax.ShapeDtypeStruct((NP, K), jnp.int32),
            jax.ShapeDtypeStruct((NP, DP), jnp.float32),
            jax.ShapeDtypeStruct((NP, H), jnp.float32),
        ],
    )(hp, ht, wa, wb, vec)


def _stats_call(uj, c):
    return pl.pallas_call(
        _stats_body,
        grid=(K, NG),
        in_specs=[
            pl.BlockSpec((NB, DP), lambda k, n: (k * NG + n, 0)),
            pl.BlockSpec((NB, H), lambda k, n: (n, 0)),
        ],
        out_specs=pl.BlockSpec((8, H), lambda k, n: (0, 0)),
        out_shape=jax.ShapeDtypeStruct((8, H), jnp.float32),
    )(uj, c)


def _out_call(uj, c, stats, vec, w2):
    return pl.pallas_call(
        _out_body,
        grid=(NG, K),
        in_specs=[
            pl.BlockSpec((NB, DP), lambda n, k: (k * NG + n, 0)),
            pl.BlockSpec((NB, H), lambda n, k: (n, 0)),
            pl.BlockSpec((8, H), lambda n, k: (0, 0)),
            pl.BlockSpec((8, H), lambda n, k: (0, 0)),
            pl.BlockSpec((H, H), lambda n, k: (0, 0)),
        ],
        out_specs=pl.BlockSpec((NB, H), lambda n, k: (n, 0)),
        out_shape=jax.ShapeDtypeStruct((N, H), jnp.float32),
    )(uj, c, stats, vec, w2)


def _mlp_call(x1, x2, x3, w1, b1, w2, b2, w3, b3, w4, b4):
    mb = 400
    grid = N // mb
    row = lambda i: (i, 0)
    cst = lambda i: (0, 0)
    return pl.pallas_call(
        _mlp_body,
        grid=(grid,),
        in_specs=[
            pl.BlockSpec((mb, H), row),
            pl.BlockSpec((mb, H), row),
            pl.BlockSpec((mb, H), row),
            pl.BlockSpec((192, 128), cst),
            pl.BlockSpec((8, 128), cst),
            pl.BlockSpec((128, 64), cst),
            pl.BlockSpec((8, 64), cst),
            pl.BlockSpec((64, 32), cst),
            pl.BlockSpec((8, 32), cst),
            pl.BlockSpec((32, 2), cst),
            pl.BlockSpec((8, 2), cst),
        ],
        out_specs=pl.BlockSpec((mb, 2), row),
        out_shape=jax.ShapeDtypeStruct((N, 2), jnp.float32),
    )(x1, x2, x3, w1, b1, w2, b2, w3, b3, w4, b4)


def _make_sc_gather():
    info = plsc.get_sparse_core_info()
    nc = info.num_cores
    nw = nc * info.num_subcores
    chunks_per_w = EPAD // (nw * CH)
    mesh = plsc.VectorSubcoreMesh(core_axis_name="c", subcore_axis_name="s")

    nbuf = 4
    groups = chunks_per_w // nbuf

    @functools.partial(
        pl.kernel, mesh=mesh,
        out_type=jax.ShapeDtypeStruct((EPAD, DP), jnp.float32),
        scratch_types=(
            [pltpu.VMEM((CH,), jnp.int32) for _ in range(nbuf)]
            + [pltpu.VMEM((CH, DP), jnp.float32) for _ in range(nbuf)]
            + [pltpu.SemaphoreType.DMA] * 3
        ),
    )
    def gather(u_hbm, idx_hbm, out_hbm, *refs):
        idx_v = refs[:nbuf]
        rows_v = refs[nbuf:2 * nbuf]
        sem_i, sem_g, sem_s = refs[2 * nbuf:]
        wid = lax.axis_index("s") * nc + lax.axis_index("c")

        def step(t, carry):
            base = (wid * chunks_per_w + t * nbuf) * CH
            hs = [pltpu.async_copy(idx_hbm.at[pl.ds(base + b * CH, CH)],
                                   idx_v[b], sem_i) for b in range(nbuf)]
            for h in hs:
                h.wait()
            hs = [pltpu.async_copy(u_hbm.at[idx_v[b]], rows_v[b], sem_g)
                  for b in range(nbuf)]
            for h in hs:
                h.wait()
            hs = [pltpu.async_copy(rows_v[b], out_hbm.at[pl.ds(base + b * CH, CH)],
                                   sem_s) for b in range(nbuf)]
            for h in hs:
                h.wait()
            return carry

        lax.fori_loop(0, groups, step, 0)

    return gather


_sc_gather_cache = []


def _sc_gather(u, idx):
    if not _sc_gather_cache:
        _sc_gather_cache.append(_make_sc_gather())
    return _sc_gather_cache[0](u, idx)


def _vec8(*rows, width=H):
    v = jnp.zeros((8, width), jnp.float32)
    for r, x in enumerate(rows):
        v = v.at[r].set(x)
    return v


def _edge_layer(h, w1, b1, g, be, w2, b2):
    d = h.shape[1]
    wa = jnp.pad(w1[:d], ((0, DP - d), (0, 0)))
    wb = jnp.pad(w1[d:], ((0, DP - d), (0, DP - H)))
    hp = jnp.pad(h, ((0, NP - N), (0, DP - d)))
    ht = hp.T
    idxp, u_p, c_p = _dist_call(hp, ht, wa, wb, _vec8(b1))
    idx_km = idxp[:N].T.reshape(-1)
    idx_pad = jnp.pad(idx_km, (0, EPAD - E))
    uj = _sc_gather(u_p, idx_pad)
    stats = _stats_call(uj, c_p)
    return _out_call(uj, c_p, stats, _vec8(g, be, b2), w2)


def kernel(x, batch, c1_W1, c1_b1, c1_g, c1_be, c1_W2, c1_b2,
           c2_W1, c2_b1, c2_g, c2_be, c2_W2, c2_b2,
           c3_W1, c3_b1, c3_g, c3_be, c3_W2, c3_b2,
           m_W1, m_b1, m_W2, m_b2, m_W3, m_b3, m_W4, m_b4):
    x1 = _edge_layer(x, c1_W1, c1_b1, c1_g, c1_be, c1_W2, c1_b2)
    x2 = _edge_layer(x1, c2_W1, c2_b1, c2_g, c2_be, c2_W2, c2_b2)
    x3 = _edge_layer(x2, c3_W1, c3_b1, c3_g, c3_be, c3_W2, c3_b2)
    return _mlp_call(x1, x2, x3,
                     m_W1, _vec8(m_b1, width=128),
                     m_W2, _vec8(m_b2, width=64),
                     m_W3, _vec8(m_b3, width=32),
                     m_W4, _vec8(m_b4, width=2))


# speedup vs baseline: 4.6237x; 1.0096x over previous
import functools

import jax
import jax.numpy as jnp
from jax import lax
from jax.experimental import pallas as pl
from jax.experimental.pallas import tpu as pltpu
from jax.experimental.pallas import tpu_sc as plsc

N = 10000
K = 16
H = 64
RB = 128
NP = 10112
DP = 128
NB = 2000
NG = N // NB
E = N * K
CH = 128
EPAD = 163840
BIGF = 3.0e38
BIGI = 1 << 30


def _dist_body(hp_ref, ht_ref, wa_ref, wb_ref, vec_ref,
               idx_ref, u_ref, c_ref):
    hb = hp_ref[...]
    ht = ht_ref[...]
    d2 = -2.0 * jnp.dot(hb, ht, preferred_element_type=jnp.float32)
    sq_row = jnp.sum(hb * hb, axis=1, keepdims=True)
    sq_col = jnp.sum(ht * ht, axis=0, keepdims=True)
    d2 = d2 + sq_row + sq_col
    col = lax.broadcasted_iota(jnp.int32, (RB, NP), 1)
    d2 = jnp.where(col < N, d2, BIGF)
    picks = []
    for _ in range(K):
        vmin = jnp.min(d2, axis=1, keepdims=True)
        ismin = d2 <= vmin
        idxk = jnp.min(jnp.where(ismin, col, BIGI), axis=1, keepdims=True)
        picks.append(idxk)
        d2 = jnp.where(col == idxk, BIGF, d2)
    idx_ref[...] = jnp.concatenate(picks, axis=1)
    u = jnp.dot(hb, wb_ref[...], preferred_element_type=jnp.float32)
    u_ref[...] = u
    c_ref[...] = (jnp.dot(hb, wa_ref[...], preferred_element_type=jnp.float32)
                  + vec_ref[0:1, :] - u[:, 0:H])


def _stats_body(uj_ref, c_ref, stats_ref):
    first = (pl.program_id(0) == 0) & (pl.program_id(1) == 0)

    @pl.when(first)
    def _():
        stats_ref[...] = jnp.zeros((8, H), jnp.float32)

    m = uj_ref[:, 0:H] + c_ref[...]
    s1 = jnp.sum(m, axis=0, keepdims=True)
    s2 = jnp.sum(m * m, axis=0, keepdims=True)
    stats_ref[0:1, :] += s1
    stats_ref[1:2, :] += s2


def _out_body(uj_ref, c_ref, stats_ref, vec_ref, w2_ref, out_ref):
    k = pl.program_id(1)
    m = uj_ref[:, 0:H] + c_ref[...]
    mu = stats_ref[0:1, :] / jnp.float32(E)
    var = stats_ref[1:2, :] / jnp.float32(E) - mu * mu
    g = vec_ref[0:1, :]
    be = vec_ref[1:2, :]
    b2 = vec_ref[2:3, :]
    mnorm = g * (m - mu) / jnp.sqrt(var + 1e-5) + be
    y = jnp.maximum(mnorm, 0.0)
    y = jnp.dot(y, w2_ref[...], preferred_element_type=jnp.float32) + b2

    @pl.when(k == 0)
    def _():
        out_ref[...] = y

    @pl.when(k > 0)
    def _():
        out_ref[...] = jnp.maximum(out_ref[...], y)


def _mlp_body(x1_ref, x2_ref, x3_ref,
              w1_ref, b1_ref, w2_ref, b2_ref,
              w3_ref, b3_ref, w4_ref, b4_ref, out_ref):
    h = jnp.concatenate([x1_ref[...], x2_ref[...], x3_ref[...]], axis=1)
    h = jnp.maximum(jnp.dot(h, w1_ref[...], preferred_element_type=jnp.float32)
                    + b1_ref[0:1, :], 0.0)
    h = jnp.maximum(jnp.dot(h, w2_ref[...], preferred_element_type=jnp.float32)
                    + b2_ref[0:1, :], 0.0)
    h = jnp.maximum(jnp.dot(h, w3_ref[...], preferred_element_type=jnp.float32)
                    + b3_ref[0:1, :], 0.0)
    z = jnp.dot(h, w4_ref[...], preferred_element_type=jnp.float32) + b4_ref[0:1, :]
    zmax = jnp.max(z, axis=1, keepdims=True)
    ez = jnp.exp(z - zmax)
    lse = jnp.log(jnp.sum(ez, axis=1, keepdims=True)) + zmax
    out_ref[...] = z - lse


def _dist_call(hp, ht, wa, wb, vec):
    grid = NP // RB
    return pl.pallas_call(
        _dist_body,
        grid=(grid,),
        in_specs=[
            pl.BlockSpec((RB, DP), lambda i: (i, 0)),
            pl.BlockSpec((DP, NP), lambda i: (0, 0)),
            pl.BlockSpec((DP, H), lambda i: (0, 0)),
            pl.BlockSpec((DP, DP), lambda i: (0, 0)),
            pl.BlockSpec((8, H), lambda i: (0, 0)),
        ],
        out_specs=[
            pl.BlockSpec((RB, K), lambda i: (i, 0)),
            pl.BlockSpec((RB, DP), lambda i: (i, 0)),
            pl.BlockSpec((RB, H), lambda i: (i, 0)),
        ],
        out_shape=[
            jax.ShapeDtypeStruct((NP, K), jnp.int32),
            jax.ShapeDtypeStruct((NP, DP), jnp.float32),
            jax.ShapeDtypeStruct((NP, H), jnp.float32),
        ],
    )(hp, ht, wa, wb, vec)


def _stats_call(uj, c):
    return pl.pallas_call(
        _stats_body,
        grid=(K, NG),
        in_specs=[
            pl.BlockSpec((NB, DP), lambda k, n: (k * NG + n, 0)),
            pl.BlockSpec((NB, H), lambda k, n: (n, 0)),
        ],
        out_specs=pl.BlockSpec((8, H), lambda k, n: (0, 0)),
        out_shape=jax.ShapeDtypeStruct((8, H), jnp.float32),
    )(uj, c)


def _out_call(uj, c, stats, vec, w2):
    return pl.pallas_call(
        _out_body,
        grid=(NG, K),
        in_specs=[
            pl.BlockSpec((NB, DP), lambda n, k: (k * NG + n, 0)),
            pl.BlockSpec((NB, H), lambda n, k: (n, 0)),
            pl.BlockSpec((8, H), lambda n, k: (0, 0)),
            pl.BlockSpec((8, H), lambda n, k: (0, 0)),
            pl.BlockSpec((H, H), lambda n, k: (0, 0)),
        ],
        out_specs=pl.BlockSpec((NB, H), lambda n, k: (n, 0)),
        out_shape=jax.ShapeDtypeStruct((N, H), jnp.float32),
    )(uj, c, stats, vec, w2)


def _mlp_call(x1, x2, x3, w1, b1, w2, b2, w3, b3, w4, b4):
    mb = 400
    grid = N // mb
    row = lambda i: (i, 0)
    cst = lambda i: (0, 0)
    return pl.pallas_call(
        _mlp_body,
        grid=(grid,),
        in_specs=[
            pl.BlockSpec((mb, H), row),
            pl.BlockSpec((mb, H), row),
            pl.BlockSpec((mb, H), row),
            pl.BlockSpec((192, 128), cst),
            pl.BlockSpec((8, 128), cst),
            pl.BlockSpec((128, 64), cst),
            pl.BlockSpec((8, 64), cst),
            pl.BlockSpec((64, 32), cst),
            pl.BlockSpec((8, 32), cst),
            pl.BlockSpec((32, 2), cst),
            pl.BlockSpec((8, 2), cst),
        ],
        out_specs=pl.BlockSpec((mb, 2), row),
        out_shape=jax.ShapeDtypeStruct((N, 2), jnp.float32),
    )(x1, x2, x3, w1, b1, w2, b2, w3, b3, w4, b4)


def _make_sc_gather():
    info = plsc.get_sparse_core_info()
    nc = info.num_cores
    nw = nc * info.num_subcores
    chunks_per_w = EPAD // (nw * CH)
    mesh = plsc.VectorSubcoreMesh(core_axis_name="c", subcore_axis_name="s")

    nbuf = 4
    groups = chunks_per_w // nbuf
    per_w = chunks_per_w * CH

    @functools.partial(
        pl.kernel, mesh=mesh,
        out_type=jax.ShapeDtypeStruct((EPAD, DP), jnp.float32),
        scratch_types=(
            [pltpu.VMEM((per_w,), jnp.int32)]
            + [pltpu.VMEM((CH, DP), jnp.float32) for _ in range(nbuf)]
            + [pltpu.SemaphoreType.DMA] * 2
        ),
    )
    def gather(u_hbm, idx_hbm, out_hbm, idx_v, *refs):
        rows_v = refs[:nbuf]
        sem_g, sem_s = refs[nbuf:]
        wid = lax.axis_index("s") * nc + lax.axis_index("c")
        base = wid * per_w
        pltpu.sync_copy(idx_hbm.at[pl.ds(base, per_w)], idx_v)

        def g_copy(t, b):
            return pltpu.make_async_copy(
                u_hbm.at[idx_v.at[pl.ds(t * CH, CH)]], rows_v[b], sem_g)

        def s_copy(t, b):
            return pltpu.make_async_copy(
                rows_v[b], out_hbm.at[pl.ds(base + t * CH, CH)], sem_s)

        for b in range(nbuf):
            g_copy(b, b).start()

        def group(g, carry):
            for b in range(nbuf):
                t = g * nbuf + b
                g_copy(t, b).wait()
                s_copy(t, b).start()
            for b in range(nbuf):
                t = g * nbuf + b
                s_copy(t, b).wait()

                @pl.when(g < groups - 1)
                def _():
                    g_copy(t + nbuf, b).start()

            return carry

        lax.fori_loop(0, groups, group, 0)

    return gather


_sc_gather_cache = []


def _sc_gather(u, idx):
    if not _sc_gather_cache:
        _sc_gather_cache.append(_make_sc_gather())
    return _sc_gather_cache[0](u, idx)


def _vec8(*rows, width=H):
    v = jnp.zeros((8, width), jnp.float32)
    for r, x in enumerate(rows):
        v = v.at[r].set(x)
    return v


def _edge_layer(h, w1, b1, g, be, w2, b2):
    d = h.shape[1]
    wa = jnp.pad(w1[:d], ((0, DP - d), (0, 0)))
    wb = jnp.pad(w1[d:], ((0, DP - d), (0, DP - H)))
    hp = jnp.pad(h, ((0, NP - N), (0, DP - d)))
    ht = hp.T
    idxp, u_p, c_p = _dist_call(hp, ht, wa, wb, _vec8(b1))
    idx_km = idxp[:N].T.reshape(-1)
    idx_pad = jnp.pad(idx_km, (0, EPAD - E))
    uj = _sc_gather(u_p, idx_pad)
    stats = _stats_call(uj, c_p)
    return _out_call(uj, c_p, stats, _vec8(g, be, b2), w2)


def kernel(x, batch, c1_W1, c1_b1, c1_g, c1_be, c1_W2, c1_b2,
           c2_W1, c2_b1, c2_g, c2_be, c2_W2, c2_b2,
           c3_W1, c3_b1, c3_g, c3_be, c3_W2, c3_b2,
           m_W1, m_b1, m_W2, m_b2, m_W3, m_b3, m_W4, m_b4):
    x1 = _edge_layer(x, c1_W1, c1_b1, c1_g, c1_be, c1_W2, c1_b2)
    x2 = _edge_layer(x1, c2_W1, c2_b1, c2_g, c2_be, c2_W2, c2_b2)
    x3 = _edge_layer(x2, c3_W1, c3_b1, c3_g, c3_be, c3_W2, c3_b2)
    return _mlp_call(x1, x2, x3,
                     m_W1, _vec8(m_b1, width=128),
                     m_W2, _vec8(m_b2, width=64),
                     m_W3, _vec8(m_b3, width=32),
                     m_W4, _vec8(m_b4, width=2))


# speedup vs baseline: 5.0599x; 1.0943x over previous
import functools

import jax
import jax.numpy as jnp
from jax import lax
from jax.experimental import pallas as pl
from jax.experimental.pallas import tpu as pltpu
from jax.experimental.pallas import tpu_sc as plsc

N = 10000
K = 16
H = 64
RB = 128
NP = 10112
DP = 128
NB = 2000
NG = N // NB
E = N * K
CH = 128
EPAD = 163840
BIGF = 3.0e38
BIGI = 1 << 30


def _dist_body(hp_ref, ht_ref, wa_ref, wb_ref, vec_ref,
               idx_ref, u_ref, c_ref):
    hb = hp_ref[...]
    ht = ht_ref[...]
    d2 = -2.0 * jnp.dot(hb, ht, preferred_element_type=jnp.float32)
    sq_row = jnp.sum(hb * hb, axis=1, keepdims=True)
    sq_col = jnp.sum(ht * ht, axis=0, keepdims=True)
    d2 = d2 + sq_row + sq_col
    col = lax.broadcasted_iota(jnp.int32, (RB, NP), 1)
    d2 = jnp.where(col < N, d2, BIGF)
    picks = []
    for _ in range(K):
        idxk = jnp.argmin(d2, axis=1).astype(jnp.int32).reshape(RB, 1)
        picks.append(idxk)
        d2 = jnp.where(col == idxk, BIGF, d2)
    idx_ref[...] = jnp.concatenate(picks, axis=1)
    u = jnp.dot(hb, wb_ref[...], preferred_element_type=jnp.float32)
    u_ref[...] = u
    c_ref[...] = (jnp.dot(hb, wa_ref[...], preferred_element_type=jnp.float32)
                  + vec_ref[0:1, :] - u[:, 0:H])


def _stats_body(uj_ref, c_ref, stats_ref):
    first = (pl.program_id(0) == 0) & (pl.program_id(1) == 0)

    @pl.when(first)
    def _():
        stats_ref[...] = jnp.zeros((8, H), jnp.float32)

    m = uj_ref[:, 0:H] + c_ref[...]
    s1 = jnp.sum(m, axis=0, keepdims=True)
    s2 = jnp.sum(m * m, axis=0, keepdims=True)
    stats_ref[0:1, :] += s1
    stats_ref[1:2, :] += s2


def _out_body(uj_ref, c_ref, stats_ref, vec_ref, w2_ref, out_ref):
    k = pl.program_id(1)
    m = uj_ref[:, 0:H] + c_ref[...]
    mu = stats_ref[0:1, :] / jnp.float32(E)
    var = stats_ref[1:2, :] / jnp.float32(E) - mu * mu
    g = vec_ref[0:1, :]
    be = vec_ref[1:2, :]
    b2 = vec_ref[2:3, :]
    mnorm = g * (m - mu) / jnp.sqrt(var + 1e-5) + be
    y = jnp.maximum(mnorm, 0.0)
    y = jnp.dot(y, w2_ref[...], preferred_element_type=jnp.float32) + b2

    @pl.when(k == 0)
    def _():
        out_ref[...] = y

    @pl.when(k > 0)
    def _():
        out_ref[...] = jnp.maximum(out_ref[...], y)


def _mlp_body(x1_ref, x2_ref, x3_ref,
              w1_ref, b1_ref, w2_ref, b2_ref,
              w3_ref, b3_ref, w4_ref, b4_ref, out_ref):
    h = jnp.concatenate([x1_ref[...], x2_ref[...], x3_ref[...]], axis=1)
    h = jnp.maximum(jnp.dot(h, w1_ref[...], preferred_element_type=jnp.float32)
                    + b1_ref[0:1, :], 0.0)
    h = jnp.maximum(jnp.dot(h, w2_ref[...], preferred_element_type=jnp.float32)
                    + b2_ref[0:1, :], 0.0)
    h = jnp.maximum(jnp.dot(h, w3_ref[...], preferred_element_type=jnp.float32)
                    + b3_ref[0:1, :], 0.0)
    z = jnp.dot(h, w4_ref[...], preferred_element_type=jnp.float32) + b4_ref[0:1, :]
    zmax = jnp.max(z, axis=1, keepdims=True)
    ez = jnp.exp(z - zmax)
    lse = jnp.log(jnp.sum(ez, axis=1, keepdims=True)) + zmax
    out_ref[...] = z - lse


def _dist_call(hp, ht, wa, wb, vec):
    grid = NP // RB
    return pl.pallas_call(
        _dist_body,
        grid=(grid,),
        in_specs=[
            pl.BlockSpec((RB, DP), lambda i: (i, 0)),
            pl.BlockSpec((DP, NP), lambda i: (0, 0)),
            pl.BlockSpec((DP, H), lambda i: (0, 0)),
            pl.BlockSpec((DP, DP), lambda i: (0, 0)),
            pl.BlockSpec((8, H), lambda i: (0, 0)),
        ],
        out_specs=[
            pl.BlockSpec((RB, K), lambda i: (i, 0)),
            pl.BlockSpec((RB, DP), lambda i: (i, 0)),
            pl.BlockSpec((RB, H), lambda i: (i, 0)),
        ],
        out_shape=[
            jax.ShapeDtypeStruct((NP, K), jnp.int32),
            jax.ShapeDtypeStruct((NP, DP), jnp.float32),
            jax.ShapeDtypeStruct((NP, H), jnp.float32),
        ],
    )(hp, ht, wa, wb, vec)


def _stats_call(uj, c):
    return pl.pallas_call(
        _stats_body,
        grid=(K, NG),
        in_specs=[
            pl.BlockSpec((NB, DP), lambda k, n: (k * NG + n, 0)),
            pl.BlockSpec((NB, H), lambda k, n: (n, 0)),
        ],
        out_specs=pl.BlockSpec((8, H), lambda k, n: (0, 0)),
        out_shape=jax.ShapeDtypeStruct((8, H), jnp.float32),
    )(uj, c)


def _out_call(uj, c, stats, vec, w2):
    return pl.pallas_call(
        _out_body,
        grid=(NG, K),
        in_specs=[
            pl.BlockSpec((NB, DP), lambda n, k: (k * NG + n, 0)),
            pl.BlockSpec((NB, H), lambda n, k: (n, 0)),
            pl.BlockSpec((8, H), lambda n, k: (0, 0)),
            pl.BlockSpec((8, H), lambda n, k: (0, 0)),
            pl.BlockSpec((H, H), lambda n, k: (0, 0)),
        ],
        out_specs=pl.BlockSpec((NB, H), lambda n, k: (n, 0)),
        out_shape=jax.ShapeDtypeStruct((N, H), jnp.float32),
    )(uj, c, stats, vec, w2)


def _mlp_call(x1, x2, x3, w1, b1, w2, b2, w3, b3, w4, b4):
    mb = 400
    grid = N // mb
    row = lambda i: (i, 0)
    cst = lambda i: (0, 0)
    return pl.pallas_call(
        _mlp_body,
        grid=(grid,),
        in_specs=[
            pl.BlockSpec((mb, H), row),
            pl.BlockSpec((mb, H), row),
            pl.BlockSpec((mb, H), row),
            pl.BlockSpec((192, 128), cst),
            pl.BlockSpec((8, 128), cst),
            pl.BlockSpec((128, 64), cst),
            pl.BlockSpec((8, 64), cst),
            pl.BlockSpec((64, 32), cst),
            pl.BlockSpec((8, 32), cst),
            pl.BlockSpec((32, 2), cst),
            pl.BlockSpec((8, 2), cst),
        ],
        out_specs=pl.BlockSpec((mb, 2), row),
        out_shape=jax.ShapeDtypeStruct((N, 2), jnp.float32),
    )(x1, x2, x3, w1, b1, w2, b2, w3, b3, w4, b4)


def _make_sc_gather():
    info = plsc.get_sparse_core_info()
    nc = info.num_cores
    nw = nc * info.num_subcores
    chunks_per_w = EPAD // (nw * CH)
    mesh = plsc.VectorSubcoreMesh(core_axis_name="c", subcore_axis_name="s")

    nbuf = 4
    groups = chunks_per_w // nbuf
    per_w = chunks_per_w * CH

    @functools.partial(
        pl.kernel, mesh=mesh,
        out_type=jax.ShapeDtypeStruct((EPAD, DP), jnp.float32),
        scratch_types=(
            [pltpu.VMEM((per_w,), jnp.int32)]
            + [pltpu.VMEM((CH, DP), jnp.float32) for _ in range(nbuf)]
            + [pltpu.SemaphoreType.DMA] * 2
        ),
    )
    def gather(u_hbm, idx_hbm, out_hbm, idx_v, *refs):
        rows_v = refs[:nbuf]
        sem_g, sem_s = refs[nbuf:]
        wid = lax.axis_index("s") * nc + lax.axis_index("c")
        base = wid * per_w
        pltpu.sync_copy(idx_hbm.at[pl.ds(base, per_w)], idx_v)

        def g_copy(t, b):
            return pltpu.make_async_copy(
                u_hbm.at[idx_v.at[pl.ds(t * CH, CH)]], rows_v[b], sem_g)

        def s_copy(t, b):
            return pltpu.make_async_copy(
                rows_v[b], out_hbm.at[pl.ds(base + t * CH, CH)], sem_s)

        for b in range(nbuf):
            g_copy(b, b).start()

        def group(g, carry):
            for b in range(nbuf):
                t = g * nbuf + b
                g_copy(t, b).wait()
                s_copy(t, b).start()
            for b in range(nbuf):
                t = g * nbuf + b
                s_copy(t, b).wait()

                @pl.when(g < groups - 1)
                def _():
                    g_copy(t + nbuf, b).start()

            return carry

        lax.fori_loop(0, groups, group, 0)

    return gather


_sc_gather_cache = []


def _sc_gather(u, idx):
    if not _sc_gather_cache:
        _sc_gather_cache.append(_make_sc_gather())
    return _sc_gather_cache[0](u, idx)


def _vec8(*rows, width=H):
    v = jnp.zeros((8, width), jnp.float32)
    for r, x in enumerate(rows):
        v = v.at[r].set(x)
    return v


def _edge_layer(h, w1, b1, g, be, w2, b2):
    d = h.shape[1]
    wa = jnp.pad(w1[:d], ((0, DP - d), (0, 0)))
    wb = jnp.pad(w1[d:], ((0, DP - d), (0, DP - H)))
    hp = jnp.pad(h, ((0, NP - N), (0, DP - d)))
    ht = hp.T
    idxp, u_p, c_p = _dist_call(hp, ht, wa, wb, _vec8(b1))
    idx_km = idxp[:N].T.reshape(-1)
    idx_pad = jnp.pad(idx_km, (0, EPAD - E))
    uj = _sc_gather(u_p, idx_pad)
    stats = _stats_call(uj, c_p)
    return _out_call(uj, c_p, stats, _vec8(g, be, b2), w2)


def kernel(x, batch, c1_W1, c1_b1, c1_g, c1_be, c1_W2, c1_b2,
           c2_W1, c2_b1, c2_g, c2_be, c2_W2, c2_b2,
           c3_W1, c3_b1, c3_g, c3_be, c3_W2, c3_b2,
           m_W1, m_b1, m_W2, m_b2, m_W3, m_b3, m_W4, m_b4):
    x1 = _edge_layer(x, c1_W1, c1_b1, c1_g, c1_be, c1_W2, c1_b2)
    x2 = _edge_layer(x1, c2_W1, c2_b1, c2_g, c2_be, c2_W2, c2_b2)
    x3 = _edge_layer(x2, c3_W1, c3_b1, c3_g, c3_be, c3_W2, c3_b2)
    return _mlp_call(x1, x2, x3,
                     m_W1, _vec8(m_b1, width=128),
                     m_W2, _vec8(m_b2, width=64),
                     m_W3, _vec8(m_b3, width=32),
                     m_W4, _vec8(m_b4, width=2))
